# Initial kernel scaffold; baseline (speedup 1.0000x reference)
import jax
import jax.numpy as jnp
from jax.experimental import pallas as pl


def kernel(volume1, volume2, volume3, xy_rel_id, yz_rel_id, xz_rel_id, flag):
    raise NotImplementedError("write your pallas kernel here")


# speedup vs baseline: 11.5425x; 11.5425x over previous
import jax
import jax.numpy as jnp
from jax import lax
from jax.experimental import pallas as pl

_B = 16384
_R = 128
_C = 128

_POS = [(0, 4, 4), (0, 6, 4), (1, 5, 5), (1, 6, 5), (2, 4, 4), (2, 5, 5),
        (2, 6, 6), (2, 7, 7), (4, 0, 4), (4, 2, 4), (5, 1, 5), (5, 2, 5),
        (6, 2, 6), (7, 2, 7)]
_NEG = [(0, 4, 1), (0, 4, 2), (0, 6, 1), (0, 6, 2), (1, 5, 0), (1, 5, 2),
        (1, 6, 0), (1, 6, 2), (2, 4, 1), (2, 4, 2), (2, 5, 0), (2, 5, 2),
        (4, 0, 1), (4, 0, 2), (4, 2, 1), (4, 2, 2), (5, 1, 0), (5, 1, 2),
        (5, 2, 0), (5, 2, 2), (2, 7, 2), (7, 2, 2)]


def _dm(cat):
    return 0 if cat < 4 else 1


def _log1mexp(x):
    return jnp.log(1.0 - jnp.exp(x))


def _loss_body(v10, v11, v20, v21, v30, v31, x0, x1, y0, y1, z0, z1, fl,
               out_ref):
    v10, v11 = v10[...], v11[...]
    v20, v21 = v20[...], v21[...]
    v30, v31 = v30[...], v31[...]
    x0, x1 = x0[...], x1[...]
    y0, y1 = y0[...], y1[...]
    z0, z1 = z0[...], z1[...]
    fl = fl[...]

    four_fl = 4 * fl
    cx = 3 - 3 * x0 - 2 * x1 + 4 * x0 * x1 + four_fl
    cy = 3 - 3 * y0 - 2 * y1 + 4 * y0 * y1 + four_fl
    cz = 3 - 3 * z0 - 2 * z1 + 4 * z0 * z1 + four_fl
    code = cx * 64 + cy * 8 + cz

    idx = (lax.broadcasted_iota(jnp.int32, (_R, _C), 0) * _C
           + lax.broadcasted_iota(jnp.int32, (_R, _C), 1))

    v1c = (v10, v11)
    v2c = (v20, v21)
    v3c = (v30, v31)

    zero = jnp.zeros((_R, _C), jnp.float32)
    pos_acc = zero
    for (xy, yz, xz) in _POS:
        t = xy * 64 + yz * 8 + xz
        w = v1c[_dm(xy)] + v2c[_dm(yz)] - v3c[_dm(xz)]
        pos_acc = pos_acc + jnp.where(code == t, w, 0.0)
    loss = -jnp.sum(pos_acc)

    big = jnp.int32(2**31 - 1)
    s12_full_1 = v10 + v11
    s12_full_2 = v20 + v21
    for (xy, yz, xz) in _NEG:
        t = xy * 64 + yz * 8 + xz
        f1, f2, f3 = _dm(xy), _dm(yz), _dm(xz)
        sel = code == t
        cnt = jnp.sum(sel.astype(jnp.int32))
        midx = jnp.where(sel, idx, big)
        p0 = jnp.min(midx)
        p1c = jnp.min(jnp.where(midx == p0, big, midx))
        p1 = jnp.where(cnt >= 2, p1c, p0)
        oh = (idx == p0, idx == p1)
        s12 = (jnp.sum(jnp.where(oh[f1], s12_full_1, zero))
               + jnp.sum(jnp.where(oh[f2], s12_full_2, zero)))
        v3a = jnp.sum(jnp.where(oh[f3], v30, zero))
        v3b = jnp.sum(jnp.where(oh[f3], v31, zero))
        lsum = s12 - (_log1mexp(v3a) + _log1mexp(v3b))
        loss = loss + jnp.where(cnt > 0, -lsum, 0.0)

    out_ref[...] = jnp.broadcast_to(loss, (1, 1))


def kernel(volume1, volume2, volume3, xy_rel_id, yz_rel_id, xz_rel_id, flag):
    shp = (_R, _C)
    args = (
        volume1[:, 0].reshape(shp), volume1[:, 1].reshape(shp),
        volume2[:, 0].reshape(shp), volume2[:, 1].reshape(shp),
        volume3[:, 0].reshape(shp), volume3[:, 1].reshape(shp),
        xy_rel_id[:, 0].astype(jnp.int32).reshape(shp),
        xy_rel_id[:, 1].astype(jnp.int32).reshape(shp),
        yz_rel_id[:, 0].astype(jnp.int32).reshape(shp),
        yz_rel_id[:, 1].astype(jnp.int32).reshape(shp),
        xz_rel_id[:, 0].astype(jnp.int32).reshape(shp),
        xz_rel_id[:, 1].astype(jnp.int32).reshape(shp),
        flag.astype(jnp.int32).reshape(shp),
    )
    out = pl.pallas_call(
        _loss_body,
        out_shape=jax.ShapeDtypeStruct((1, 1), jnp.float32),
    )(*args)
    return out[0, 0]
